# Initial kernel scaffold; baseline (speedup 1.0000x reference)
import jax
import jax.numpy as jnp
from jax.experimental import pallas as pl


def kernel(xyz, cls_label, params):
    raise NotImplementedError("write your pallas kernel here")


# speedup vs baseline: 1.1297x; 1.1297x over previous
import jax
import jax.numpy as jnp
from jax import lax
from jax.experimental import pallas as pl

_INTERPRET = False
F32 = jnp.float32
I32 = jnp.int32


def _mm(x, wt, bias=None, stats=False, tm=1024):
    B, M, Cin = x.shape
    Cout = wt.shape[1]
    tm = min(tm, M)
    assert M % tm == 0
    have_bias = bias is not None

    def body(*refs):
        if have_bias:
            x_ref, wt_ref, b_ref = refs[:3]
            orefs = refs[3:]
        else:
            x_ref, wt_ref = refs[:2]
            orefs = refs[2:]
        y = jnp.dot(x_ref[0], wt_ref[...], preferred_element_type=F32)
        if have_bias:
            y = y + b_ref[...]
        orefs[0][0] = y

    in_specs = [
        pl.BlockSpec((1, tm, Cin), lambda bb, m: (bb, m, 0)),
        pl.BlockSpec((Cin, Cout), lambda bb, m: (0, 0)),
    ]
    args = [x, wt]
    if have_bias:
        in_specs.append(pl.BlockSpec((1, Cout), lambda bb, m: (0, 0)))
        args.append(bias.reshape(1, Cout))
    res = pl.pallas_call(
        body,
        grid=(B, M // tm),
        in_specs=in_specs,
        out_specs=pl.BlockSpec((1, tm, Cout), lambda bb, m: (bb, m, 0)),
        out_shape=jax.ShapeDtypeStruct((B, M, Cout), F32),
        interpret=_INTERPRET,
    )(*args)
    return res


def _norm(y, st_ref, g_ref, be_ref, cnt):
    st = st_ref[...]
    mean = st[0:1, :]
    var = st[1:2, :]
    xn = (y - mean) / jnp.sqrt(var + 1e-5)
    return jnp.maximum(xn * g_ref[...] + be_ref[...], 0.0)


def _bnrelu_mm(y, st, cnt, g, be, wt, bias=None, tm=1024):
    B, M, Cin = y.shape
    Cout = wt.shape[1]
    tm = min(tm, M)
    assert M % tm == 0
    have_bias = bias is not None

    def body(*refs):
        y_ref, st_ref, g_ref, be_ref, wt_ref = refs[:5]
        idx = 5
        if have_bias:
            b_ref = refs[idx]
            idx += 1
        o_ref = refs[idx]
        a = _norm(y_ref[0], st_ref, g_ref, be_ref, cnt)
        y2 = jnp.dot(a, wt_ref[...], preferred_element_type=F32)
        if have_bias:
            y2 = y2 + b_ref[...]
        o_ref[0] = y2

    in_specs = [
        pl.BlockSpec((1, tm, Cin), lambda bb, m: (bb, m, 0)),
        pl.BlockSpec((2, Cin), lambda bb, m: (0, 0)),
        pl.BlockSpec((1, Cin), lambda bb, m: (0, 0)),
        pl.BlockSpec((1, Cin), lambda bb, m: (0, 0)),
        pl.BlockSpec((Cin, Cout), lambda bb, m: (0, 0)),
    ]
    args = [y, st, g.reshape(1, Cin), be.reshape(1, Cin), wt]
    if have_bias:
        in_specs.append(pl.BlockSpec((1, Cout), lambda bb, m: (0, 0)))
        args.append(bias.reshape(1, Cout))
    return pl.pallas_call(
        body,
        grid=(B, M // tm),
        in_specs=in_specs,
        out_specs=pl.BlockSpec((1, tm, Cout), lambda bb, m: (bb, m, 0)),
        out_shape=jax.ShapeDtypeStruct((B, M, Cout), F32),
        interpret=_INTERPRET,
    )(*args)


def _bnrelu_max(y, st, cnt, g, be, S, ns, ts=1):
    B, M, C = y.shape
    y4 = y.reshape(B, S, ns, C)
    ts = min(ts, S)
    assert S % ts == 0

    def body(y_ref, st_ref, g_ref, be_ref, o_ref):
        a = _norm(y_ref[0].reshape(ts * ns, C), st_ref, g_ref, be_ref, cnt)
        o_ref[0] = jnp.max(a.reshape(ts, ns, C), axis=1)

    return pl.pallas_call(
        body,
        grid=(B, S // ts),
        in_specs=[
            pl.BlockSpec((1, ts, ns, C), lambda bb, s: (bb, s, 0, 0)),
            pl.BlockSpec((2, C), lambda bb, s: (0, 0)),
            pl.BlockSpec((1, C), lambda bb, s: (0, 0)),
            pl.BlockSpec((1, C), lambda bb, s: (0, 0)),
        ],
        out_specs=pl.BlockSpec((1, ts, C), lambda bb, s: (bb, s, 0)),
        out_shape=jax.ShapeDtypeStruct((B, S, C), F32),
        interpret=_INTERPRET,
    )(y4, st, g.reshape(1, C), be.reshape(1, C))


def _fp_mm(p1, wnn, q, wp1t, bias, cls=None, wct=None, tn=512):
    B, N, C1 = p1.shape
    S = wnn.shape[2]
    Cout = wp1t.shape[1]
    tn = min(tn, N)
    assert N % tn == 0
    have_cls = cls is not None

    def body(*refs):
        p1_ref, wnn_ref, q_ref, wp1_ref, b_ref = refs[:5]
        idx = 5
        if have_cls:
            cls_ref, wct_ref = refs[idx : idx + 2]
            idx += 2
        y_ref = refs[idx]
        y = jnp.dot(p1_ref[0], wp1_ref[...], preferred_element_type=F32)
        y = y + jnp.dot(wnn_ref[0], q_ref[0], preferred_element_type=F32)
        y = y + b_ref[...]
        if have_cls:
            y = y + jnp.dot(cls_ref[0], wct_ref[...], preferred_element_type=F32)
        y_ref[0] = y

    in_specs = [
        pl.BlockSpec((1, tn, C1), lambda bb, n: (bb, n, 0)),
        pl.BlockSpec((1, tn, S), lambda bb, n: (bb, n, 0)),
        pl.BlockSpec((1, S, Cout), lambda bb, n: (bb, 0, 0)),
        pl.BlockSpec((C1, Cout), lambda bb, n: (0, 0)),
        pl.BlockSpec((1, Cout), lambda bb, n: (0, 0)),
    ]
    args = [p1, wnn, q, wp1t, bias.reshape(1, Cout)]
    if have_cls:
        in_specs.append(pl.BlockSpec((1, 1, 16), lambda bb, n: (bb, 0, 0)))
        in_specs.append(pl.BlockSpec((16, Cout), lambda bb, n: (0, 0)))
        args.append(cls)
        args.append(wct)
    return pl.pallas_call(
        body,
        grid=(B, N // tn),
        in_specs=in_specs,
        out_specs=pl.BlockSpec((1, tn, Cout), lambda bb, n: (bb, n, 0)),
        out_shape=jax.ShapeDtypeStruct((B, N, Cout), F32),
        interpret=_INTERPRET,
    )(*args)


def _fp3_mm(p1, p2row, wp1t, wintt, bias):
    B, M, C1 = p1.shape
    C2 = p2row.shape[2]
    Cout = wp1t.shape[1]

    def body(p1_ref, p2_ref, wp1_ref, wint_ref, b_ref, y_ref):
        c = jnp.dot(p2_ref[0], wint_ref[...], preferred_element_type=F32)
        y = jnp.dot(p1_ref[0], wp1_ref[...], preferred_element_type=F32)
        y_ref[0] = y + c + b_ref[...]

    return pl.pallas_call(
        body,
        grid=(B,),
        in_specs=[
            pl.BlockSpec((1, M, C1), lambda bb: (bb, 0, 0)),
            pl.BlockSpec((1, 1, C2), lambda bb: (bb, 0, 0)),
            pl.BlockSpec((C1, Cout), lambda bb: (0, 0)),
            pl.BlockSpec((C2, Cout), lambda bb: (0, 0)),
            pl.BlockSpec((1, Cout), lambda bb: (0, 0)),
        ],
        out_specs=pl.BlockSpec((1, M, Cout), lambda bb: (bb, 0, 0)),
        out_shape=jax.ShapeDtypeStruct((B, M, Cout), F32),
        interpret=_INTERPRET,
    )(p1, p2row, wp1t, wintt, bias.reshape(1, Cout))


def _final(y, st, cnt, g, be, w2t, b2, tn=512):
    B, N, C = y.shape
    Cout = w2t.shape[1]
    tn = min(tn, N)
    assert N % tn == 0

    def body(y_ref, st_ref, g_ref, be_ref, w_ref, b_ref, o_ref):
        a = _norm(y_ref[0], st_ref, g_ref, be_ref, cnt)
        x = jnp.dot(a, w_ref[...], preferred_element_type=F32) + b_ref[...]
        m = jnp.max(x, axis=1, keepdims=True)
        sh = x - m
        o_ref[0] = sh - jnp.log(jnp.sum(jnp.exp(sh), axis=1, keepdims=True))

    return pl.pallas_call(
        body,
        grid=(B, N // tn),
        in_specs=[
            pl.BlockSpec((1, tn, C), lambda bb, n: (bb, n, 0)),
            pl.BlockSpec((2, C), lambda bb, n: (0, 0)),
            pl.BlockSpec((1, C), lambda bb, n: (0, 0)),
            pl.BlockSpec((1, C), lambda bb, n: (0, 0)),
            pl.BlockSpec((C, Cout), lambda bb, n: (0, 0)),
            pl.BlockSpec((1, Cout), lambda bb, n: (0, 0)),
        ],
        out_specs=pl.BlockSpec((1, tn, Cout), lambda bb, n: (bb, n, 0)),
        out_shape=jax.ShapeDtypeStruct((B, N, Cout), F32),
        interpret=_INTERPRET,
    )(y, st, g.reshape(1, C), be.reshape(1, C), w2t, b2.reshape(1, Cout))


def _stats(y):
    return jnp.stack([jnp.mean(y, axis=(0, 1)), jnp.var(y, axis=(0, 1))])


def _fps(xyz, npoint):
    B, _, N = xyz.shape
    xt = xyz.transpose(0, 2, 1)

    def body(i, state):
        cent, dist, far = state
        cent = cent.at[:, i].set(far)
        c = xt[jnp.arange(B), far, :][:, None, :]
        d = jnp.sum((xt - c) ** 2, -1)
        dist = jnp.minimum(dist, d)
        far = jnp.argmax(dist, -1).astype(I32)
        return (cent, dist, far)

    cent0 = jnp.zeros((B, npoint), I32)
    dist0 = jnp.full((B, N), 1e10, xt.dtype)
    far0 = jnp.zeros((B,), I32)
    cent, _, _ = lax.fori_loop(0, npoint, body, (cent0, dist0, far0))
    return jax.vmap(lambda p, i: p[i])(xt, cent)


def _sqd(a, b):
    d = -2.0 * jnp.matmul(a, b.transpose(0, 2, 1))
    d = d + jnp.sum(a ** 2, -1)[:, :, None]
    d = d + jnp.sum(b ** 2, -1)[:, None, :]
    return d


def _select(sq, r2, ns):
    B, S, N = sq.shape
    gi = jnp.broadcast_to(jnp.arange(N, dtype=I32)[None, None, :], (B, S, N))
    gi = jnp.where(sq > r2, N, gi)
    gi = jnp.sort(gi, axis=-1)[:, :, :ns]
    first = jnp.broadcast_to(gi[:, :, :1], gi.shape)
    return jnp.where(gi == N, first, gi)


def _sa_ring(z_in, nxr, sq, blk, radius, ns, S):
    B, N, Cin = z_in.shape
    gi = _select(sq, radius * radius, ns)
    gp = jax.vmap(lambda p, i: p[i])(z_in, gi.reshape(B, S * ns))
    gp = gp.reshape(B, S, ns, Cin)
    ctr = jnp.concatenate([jnp.zeros((B, S, Cin - 3), F32), nxr], axis=-1)
    gp = gp - ctr[:, :, None, :]
    y = gp
    for p in blk:
        y = jnp.einsum('oc,bsnc->bsno', p['w'], y) + p['b']
        m = jnp.mean(y, axis=(0, 1, 2), keepdims=True)
        v = jnp.var(y, axis=(0, 1, 2), keepdims=True)
        y = (y - m) / jnp.sqrt(v + 1e-5)
        y = jax.nn.relu(y * p['g'] + p['be'])
    return jnp.max(y, axis=2)


def _knn3(x1, x2):
    B, N, _ = x1.shape
    S = x2.shape[1]
    dists = _sqd(x1, x2)
    idx = jnp.argsort(dists, axis=-1)[:, :, :3]
    d3 = jnp.take_along_axis(dists, idx, axis=-1)
    rec = 1.0 / (d3 + 1e-8)
    w = rec / jnp.sum(rec, axis=2, keepdims=True)
    wnn = jnp.zeros((B, N, S), F32)
    bi = jnp.arange(B)[:, None, None]
    ni = jnp.arange(N)[None, :, None]
    return wnn.at[bi, ni, idx].set(w)


def _forward(xyz, cls_label, params, S1=512, S2=128,
             ns1=(32, 64, 128), ns2=(64, 128)):
    B, _, N = xyz.shape
    xr = xyz.transpose(0, 2, 1)

    nx1 = _fps(xyz, S1)
    sq1 = _sqd(nx1, xr)
    z_in1 = jnp.concatenate([xr, xr], axis=-1)
    outs1 = [
        _sa_ring(z_in1, nx1, sq1, blk, r, ns, S1)
        for blk, r, ns in zip(params["sa1"], [0.1, 0.2, 0.4], ns1)
    ]
    l1_points = jnp.concatenate(outs1, axis=-1)

    nx2 = _fps(nx1.transpose(0, 2, 1), S2)
    sq2 = _sqd(nx2, nx1)
    z_in2 = jnp.concatenate([l1_points, nx1], axis=-1)
    outs2 = [
        _sa_ring(z_in2, nx2, sq2, blk, r, ns, S2)
        for blk, r, ns in zip(params["sa2"], [0.4, 0.8], ns2)
    ]
    l2_points = jnp.concatenate(outs2, axis=-1)

    blk3 = params["sa3"]
    cnt3 = B * S2
    x3 = jnp.concatenate([nx2, l2_points], axis=-1)
    y = _mm(x3, blk3[0]["w"].T, blk3[0]["b"])
    st = _stats(y)
    y = _bnrelu_mm(y, st, cnt3, blk3[0]["g"], blk3[0]["be"],
                   blk3[1]["w"].T, blk3[1]["b"])
    st2 = _stats(y)
    y = _bnrelu_mm(y, st2, cnt3, blk3[1]["g"], blk3[1]["be"],
                   blk3[2]["w"].T, blk3[2]["b"])
    st3 = _stats(y)
    l3_points = _bnrelu_max(y, st3, cnt3, blk3[2]["g"], blk3[2]["be"], 1, S2,
                            ts=1)

    f3 = params["fp3"]
    w = f3[0]["w"]
    y = _fp3_mm(l2_points, l3_points, w[:, :512].T, w[:, 512:].T, f3[0]["b"])
    st = _stats(y)
    y = _bnrelu_mm(y, st, cnt3, f3[0]["g"], f3[0]["be"],
                   f3[1]["w"].T, f3[1]["b"])
    st2 = _stats(y)

    f2 = params["fp2"]
    w = f2[0]["w"]
    wnn2 = _knn3(nx1, nx2)
    q2 = _bnrelu_mm(y, st2, cnt3, f3[1]["g"], f3[1]["be"],
                    w[:, 320:].T)
    cnt2 = B * S1
    y = _fp_mm(l1_points, wnn2, q2, w[:, :320].T, f2[0]["b"])
    st = _stats(y)
    y = _bnrelu_mm(y, st, cnt2, f2[0]["g"], f2[0]["be"],
                   f2[1]["w"].T, f2[1]["b"])
    st2 = _stats(y)

    f1 = params["fp1"]
    w = f1[0]["w"]
    wnn1 = _knn3(xr, nx1)
    q1 = _bnrelu_mm(y, st2, cnt2, f2[1]["g"], f2[1]["be"],
                    w[:, 22:].T)
    cnt0 = B * N
    wp1t = (w[:, 16:19] + w[:, 19:22]).T
    y = _fp_mm(xr, wnn1, q1, wp1t, f1[0]["b"],
               cls=cls_label.reshape(B, 1, 16), wct=w[:, :16].T)
    st = _stats(y)
    y = _bnrelu_mm(y, st, cnt0, f1[0]["g"], f1[0]["be"],
                   f1[1]["w"].T, f1[1]["b"])
    st2 = _stats(y)

    c1 = params["conv1"][0]
    y = _bnrelu_mm(y, st2, cnt0, f1[1]["g"], f1[1]["be"],
                   c1["w"].T, c1["b"])
    st = _stats(y)
    return _final(y, st, cnt0, c1["g"], c1["be"],
                  params["conv2_w"].T, params["conv2_b"])


def kernel(xyz, cls_label, params):
    return _forward(xyz, cls_label, params)
